# baseline (speedup 1.0000x reference)
import functools

import jax
import jax.numpy as jnp
from jax import lax
from jax.experimental import pallas as pl
from jax.experimental.pallas import tpu as pltpu
from jax.experimental.pallas import tpu_sc as plsc

B, T, D = 4, 4096, 4096
E = 64
K = 8
COEF = 0.01
S = B * T

NCH = 2
SC_TOK = S // NCH
TS = 1024
GRID = SC_TOK // TS

NW = 32
TOK_W = SC_TOK // NW
CH = TOK_W * E


def _proj_body(x_ref, w_ref, lg_ref, zsq_ref, p_ref, c_ref):
    i = pl.program_id(0)

    @pl.when(i == 0)
    def _init():
        zsq_ref[...] = jnp.zeros_like(zsq_ref)
        p_ref[...] = jnp.zeros_like(p_ref)
        c_ref[...] = jnp.zeros_like(c_ref)

    logits = jax.lax.dot_general(
        x_ref[...], w_ref[...],
        dimension_numbers=(((1,), (1,)), ((), ())),
        preferred_element_type=jnp.float32,
        precision=jax.lax.Precision.DEFAULT)
    lg_ref[...] = logits

    lane = jax.lax.broadcasted_iota(jnp.int32, (TS, E), 1)
    m0 = jnp.max(logits, axis=-1, keepdims=True)
    top1 = jnp.min(jnp.where(logits == m0, lane, E), axis=-1,
                   keepdims=True)

    pe = jnp.exp(logits - m0)
    se = jnp.sum(pe, axis=-1, keepdims=True)
    log_z = m0 + jnp.log(se)
    zsq_ref[...] += jnp.sum(log_z * log_z).reshape(1, 1)
    p_ref[...] += jnp.sum(pe / se, axis=0).reshape(1, E)
    c_ref[...] += jnp.sum(
        (top1 == lane).astype(jnp.float32), axis=0).reshape(1, E)


def _tc_project(x_chunk, W):
    out_shapes = (
        jax.ShapeDtypeStruct((SC_TOK, E), jnp.float32),
        jax.ShapeDtypeStruct((1, 1), jnp.float32),
        jax.ShapeDtypeStruct((1, E), jnp.float32),
        jax.ShapeDtypeStruct((1, E), jnp.float32),
    )
    grid_spec = pl.GridSpec(
        grid=(GRID,),
        in_specs=[
            pl.BlockSpec((TS, D), lambda i: (i, 0)),
            pl.BlockSpec((E, D), lambda i: (0, 0)),
        ],
        out_specs=(
            pl.BlockSpec((TS, E), lambda i: (i, 0)),
            pl.BlockSpec((1, 1), lambda i: (0, 0)),
            pl.BlockSpec((1, E), lambda i: (0, 0)),
            pl.BlockSpec((1, E), lambda i: (0, 0)),
        ),
    )
    return pl.pallas_call(
        _proj_body,
        grid_spec=grid_spec,
        out_shape=out_shapes,
        compiler_params=pltpu.CompilerParams(
            dimension_semantics=("arbitrary",)),
    )(x_chunk, W)


def _merge16(ka, va, kb, vb):
    sa = plsc.sort_key_val(ka, va, descending=True)
    sb = plsc.sort_key_val(kb, vb, descending=False)
    m = sa[0] >= sb[0]
    return jnp.where(m, sa[0], sb[0]), jnp.where(m, sa[1], sb[1])


@functools.partial(
    pl.kernel,
    out_type=(
        jax.ShapeDtypeStruct((SC_TOK * K,), jnp.int32),
        jax.ShapeDtypeStruct((SC_TOK * K,), jnp.float32),
    ),
    mesh=plsc.VectorSubcoreMesh(core_axis_name="c", subcore_axis_name="s"),
    scratch_types=[
        pltpu.VMEM((CH,), jnp.float32),
        pltpu.VMEM((TOK_W * K + 8,), jnp.int32),
        pltpu.VMEM((TOK_W * K + 8,), jnp.float32),
    ],
    compiler_params=pltpu.CompilerParams(needs_layout_passes=False),
)
def _route_sc(lg_hbm, idx_hbm, gate_hbm, lg_v, idx_v, gate_v):
    wid = lax.axis_index("s") * 2 + lax.axis_index("c")
    pltpu.sync_copy(lg_hbm.at[pl.ds(wid * CH, CH)], lg_v)

    lane = lax.iota(jnp.int32, 16)
    fhalf = lane < 8

    @plsc.parallel_loop(0, TOK_W, unroll=4)
    def body(t):
        o = t * E
        k0 = lg_v[pl.ds(o, 16)]
        k1 = lg_v[pl.ds(o + 16, 16)]
        k2 = lg_v[pl.ds(o + 32, 16)]
        k3 = lg_v[pl.ds(o + 48, 16)]
        ka, va = _merge16(k0, lane, k1, lane + 16)
        kb, vb = _merge16(k2, lane + 32, k3, lane + 48)
        kt, vt = _merge16(ka, va, kb, vb)
        sk = plsc.sort_key_val(kt, vt, descending=True)
        keys, vals = sk[0], sk[1]
        m0 = jnp.max(keys)
        e = jnp.where(fhalf, jnp.exp(keys - m0), 0.0)
        g = e / jnp.sum(e)
        plsc.store_compressed(idx_v.at[pl.ds(t * K, 16)], vals, mask=fhalf)
        plsc.store_compressed(gate_v.at[pl.ds(t * K, 16)], g, mask=fhalf)
    ob = wid * TOK_W * K
    pltpu.sync_copy(idx_v.at[pl.ds(0, TOK_W * K)],
                    idx_hbm.at[pl.ds(ob, TOK_W * K)])
    pltpu.sync_copy(gate_v.at[pl.ds(0, TOK_W * K)],
                    gate_hbm.at[pl.ds(ob, TOK_W * K)])


@jax.jit
def kernel(x, W):
    x_flat = x.reshape(S, D)
    idxs, gates, zsqs, ps, cs = [], [], [], [], []
    for n in range(NCH):
        xc = lax.slice_in_dim(x_flat, n * SC_TOK, (n + 1) * SC_TOK, axis=0)
        logits, zsq, p, c = _tc_project(xc, W)
        idx_flat, gate_flat = _route_sc(logits.reshape(SC_TOK * E))
        idxs.append(idx_flat.reshape(SC_TOK, K))
        gates.append(gate_flat.reshape(SC_TOK, K))
        zsqs.append(zsq)
        ps.append(p)
        cs.append(c)
    zsq = sum(zsqs)[0, 0]
    p = sum(ps)
    c = sum(cs)
    aux = COEF * (zsq / S) + COEF * E * jnp.sum((c / S) * (p / S))
    return (jnp.concatenate(idxs, axis=0),
            jnp.concatenate(gates, axis=0),
            aux)


# speedup vs baseline: 2.1825x; 2.1825x over previous
import functools

import jax
import jax.numpy as jnp
from jax import lax
from jax.experimental import pallas as pl
from jax.experimental.pallas import tpu as pltpu
from jax.experimental.pallas import tpu_sc as plsc

B, T, D = 4, 4096, 4096
E = 64
K = 8
COEF = 0.01
S = B * T

NCH = 2
SC_TOK = S // NCH
TS = 1024
GRID = SC_TOK // TS

NW = 32
TOK_W = SC_TOK // NW
CH = TOK_W * E


def _proj_body(x_ref, w_ref, lg_ref, zsq_ref, p_ref, c_ref):
    i = pl.program_id(0)

    @pl.when(i == 0)
    def _init():
        zsq_ref[...] = jnp.zeros_like(zsq_ref)
        p_ref[...] = jnp.zeros_like(p_ref)
        c_ref[...] = jnp.zeros_like(c_ref)

    logits = jax.lax.dot_general(
        x_ref[...], w_ref[...],
        dimension_numbers=(((1,), (1,)), ((), ())),
        preferred_element_type=jnp.float32,
        precision=jax.lax.Precision.DEFAULT)
    lg_ref[...] = logits

    lane = jax.lax.broadcasted_iota(jnp.int32, (TS, E), 1)
    m0 = jnp.max(logits, axis=-1, keepdims=True)
    top1 = jnp.min(jnp.where(logits == m0, lane, E), axis=-1,
                   keepdims=True)

    pe = jnp.exp(logits - m0)
    se = jnp.sum(pe, axis=-1, keepdims=True)
    log_z = m0 + jnp.log(se)
    zsq_ref[...] += jnp.sum(log_z * log_z).reshape(1, 1)
    p_ref[...] += jnp.sum(pe / se, axis=0).reshape(1, E)
    c_ref[...] += jnp.sum(
        (top1 == lane).astype(jnp.float32), axis=0).reshape(1, E)


def _tc_project(x_flat, W, n):
    out_shapes = (
        jax.ShapeDtypeStruct((SC_TOK, E), jnp.float32),
        jax.ShapeDtypeStruct((1, 1), jnp.float32),
        jax.ShapeDtypeStruct((1, E), jnp.float32),
        jax.ShapeDtypeStruct((1, E), jnp.float32),
    )
    grid_spec = pl.GridSpec(
        grid=(GRID,),
        in_specs=[
            pl.BlockSpec((TS, D), lambda i, n=n: (n * GRID + i, 0)),
            pl.BlockSpec((E, D), lambda i: (0, 0)),
        ],
        out_specs=(
            pl.BlockSpec((TS, E), lambda i: (i, 0)),
            pl.BlockSpec((1, 1), lambda i: (0, 0)),
            pl.BlockSpec((1, E), lambda i: (0, 0)),
            pl.BlockSpec((1, E), lambda i: (0, 0)),
        ),
    )
    return pl.pallas_call(
        _proj_body,
        grid_spec=grid_spec,
        out_shape=out_shapes,
        compiler_params=pltpu.CompilerParams(
            dimension_semantics=("arbitrary",)),
    )(x_flat, W)


def _merge16(ka, va, kb, vb):
    sa = plsc.sort_key_val(ka, va, descending=True)
    sb = plsc.sort_key_val(kb, vb, descending=False)
    m = sa[0] >= sb[0]
    return jnp.where(m, sa[0], sb[0]), jnp.where(m, sa[1], sb[1])


@functools.partial(
    pl.kernel,
    out_type=(
        jax.ShapeDtypeStruct((SC_TOK * K,), jnp.int32),
        jax.ShapeDtypeStruct((SC_TOK * K,), jnp.float32),
    ),
    mesh=plsc.VectorSubcoreMesh(core_axis_name="c", subcore_axis_name="s"),
    scratch_types=[
        pltpu.VMEM((CH,), jnp.float32),
        pltpu.VMEM((TOK_W * K + 8,), jnp.int32),
        pltpu.VMEM((TOK_W * K + 8,), jnp.float32),
    ],
    compiler_params=pltpu.CompilerParams(needs_layout_passes=False),
)
def _route_sc(lg_hbm, idx_hbm, gate_hbm, lg_v, idx_v, gate_v):
    wid = lax.axis_index("s") * 2 + lax.axis_index("c")
    pltpu.sync_copy(lg_hbm.at[pl.ds(wid * CH, CH)], lg_v)

    lane = lax.iota(jnp.int32, 16)
    fhalf = lane < 8

    @plsc.parallel_loop(0, TOK_W, unroll=4)
    def body(t):
        o = t * E
        k0 = lg_v[pl.ds(o, 16)]
        k1 = lg_v[pl.ds(o + 16, 16)]
        k2 = lg_v[pl.ds(o + 32, 16)]
        k3 = lg_v[pl.ds(o + 48, 16)]
        ka, va = _merge16(k0, lane, k1, lane + 16)
        kb, vb = _merge16(k2, lane + 32, k3, lane + 48)
        kt, vt = _merge16(ka, va, kb, vb)
        sk = plsc.sort_key_val(kt, vt, descending=True)
        keys, vals = sk[0], sk[1]
        m0 = jnp.max(keys)
        e = jnp.where(fhalf, jnp.exp(keys - m0), 0.0)
        g = e / jnp.sum(e)
        plsc.store_compressed(idx_v.at[pl.ds(t * K, 16)], vals, mask=fhalf)
        plsc.store_compressed(gate_v.at[pl.ds(t * K, 16)], g, mask=fhalf)
    ob = wid * TOK_W * K
    pltpu.sync_copy(idx_v.at[pl.ds(0, TOK_W * K)],
                    idx_hbm.at[pl.ds(ob, TOK_W * K)])
    pltpu.sync_copy(gate_v.at[pl.ds(0, TOK_W * K)],
                    gate_hbm.at[pl.ds(ob, TOK_W * K)])


@jax.jit
def kernel(x, W):
    x_flat = x.reshape(S, D)
    idxs, gates, zsqs, ps, cs = [], [], [], [], []
    for n in range(NCH):
        logits, zsq, p, c = _tc_project(x_flat, W, n)
        idx_flat, gate_flat = _route_sc(logits.reshape(SC_TOK * E))
        idxs.append(idx_flat.reshape(SC_TOK, K))
        gates.append(gate_flat.reshape(SC_TOK, K))
        zsqs.append(zsq)
        ps.append(p)
        cs.append(c)
    zsq = sum(zsqs)[0, 0]
    p = sum(ps)
    c = sum(cs)
    aux = COEF * (zsq / S) + COEF * E * jnp.sum((c / S) * (p / S))
    return (jnp.concatenate(idxs, axis=0),
            jnp.concatenate(gates, axis=0),
            aux)
